# Initial kernel scaffold; baseline (speedup 1.0000x reference)
import jax
import jax.numpy as jnp
from jax.experimental import pallas as pl


def kernel(random_vector, params):
    raise NotImplementedError("write your pallas kernel here")


# speedup vs baseline: 7.0553x; 7.0553x over previous
from functools import partial

import jax
import jax.numpy as jnp
from jax.experimental import pallas as pl
from jax.experimental.pallas import tpu as pltpu

_N_EVENTS = 512
_N_FEAT_OUT = 3
_F = 32
_G = 8
_LEVELS = 8
_POST = 2
_EB = 64

_PRE_AB = [(_F, _F)] * 5
_PROJ_AB = [(_F + _G, _F + _G), (_F + _G, 2 * _F)] + [(2 * _F, 2 * _F)] * 3
_GIN_AB = [(_F + _G, _F + _G), (_F + _G, _F)] + [(_F, _F)] * 3


def _leaky(x):
    return jnp.where(x >= 0.0, x, 0.2 * x)


def _dot(x, w):
    return jnp.dot(x, w, preferred_element_type=jnp.float32)


def _pmlp_rest(layers, x, p):
    for W4, b4, a, b in layers:
        x = _leaky(_dot(x, W4[:p * a, :p * b]) + b4[:, :p * b])
    return x


def _pexpand(v, k, e):
    p = min(2 ** k, 4)
    if p > 1:
        v = jnp.concatenate([v] * p, axis=1)
    if k >= 3:
        r = 2 ** (k - 2)
        c = v.shape[1]
        v = jnp.broadcast_to(v[:, None, :], (e, r, c)).reshape(e * r, c)
    return v


def _fold(h, p):
    acc = h[:, :_F]
    for i in range(1, p):
        acc = acc + h[:, _F * i:_F * (i + 1)]
    return acc


def _dyn_hlvs(xs, pre, post, e):
    s = None
    for k, xk in enumerate(xs):
        p = min(2 ** k, 4)
        h = _pmlp_rest(pre, xk, p)
        if k >= 3:
            h = h.reshape(e, 2 ** (k - 2), 4 * _F).sum(axis=1)
        hk = _fold(h, p)
        s = hk if s is None else s + hk
    cnt = float(2 ** len(xs) - 1)
    means = s / cnt
    W0, b0 = post[0][0], post[0][1]
    t = _leaky(_dot(means, W0[:_F]) + _dot(s, W0[_F:]) + b0)
    for W, b, _, _ in post[1:]:
        t = _leaky(_dot(t, W) + b)
    return t


def _dup2(prev, p):
    if p == 1:
        return jnp.concatenate([prev, prev], axis=1)
    if p == 2:
        a, b = prev[:, :_F], prev[:, _F:]
        return jnp.concatenate([a, a, b, b], axis=1)
    gs = [prev[:, _F * i:_F * (i + 1)] for i in range(4)]
    d = jnp.concatenate(
        [gs[0], gs[0], gs[1], gs[1], gs[2], gs[2], gs[3], gs[3]], axis=1)
    return d.reshape(2 * prev.shape[0], 4 * _F)


def _gin(xs, g, ginWx4, ginWg, ginb4, ginL, e):
    gg = _dot(g, ginWg)
    new = []
    for k, xk in enumerate(xs):
        p = min(2 ** k, 4)
        if k == 0:
            u = xk
            gt = gg
        else:
            u = xk + _dup2(xs[k - 1], min(2 ** (k - 1), 4))
            gt = 2.0 * _pexpand(gg, k, e)
        w = _F + _G
        t = _leaky(_dot(u, ginWx4[:_F * p, :w * p]) + gt + ginb4[:, :w * p])
        new.append(_pmlp_rest(ginL, t, p))
    return new


def _tree_kernel(x0_ref, *refs):
    it = iter(refs[:42])

    def layers(n, ab):
        out = []
        for i in range(n):
            W = next(it)[...]
            b = next(it)[...]
            out.append((W, b, ab[i][0], ab[i][1]))
        return out

    pre = layers(5, _PRE_AB)
    post = layers(5, [(0, 0)] * 5)
    projWx4 = next(it)[...]
    projWg = next(it)[...]
    projb4 = next(it)[...]
    projL = layers(4, _PROJ_AB[1:])
    ginWx4 = next(it)[...]
    ginWg = next(it)[...]
    ginb4 = next(it)[...]
    ginL = layers(4, _GIN_AB[1:])
    outs = refs[42:]
    e = x0_ref.shape[0]

    xs = [x0_ref[...]]
    for l in range(_LEVELS - 1):
        g = _dyn_hlvs(xs, pre, post, e)
        p = min(2 ** l, 4)
        w = _F + _G
        gp = _pexpand(_dot(g, projWg), l, e)
        t = _leaky(_dot(xs[l], projWx4[:_F * p, :w * p]) + gp
                   + projb4[:, :w * p])
        t = _pmlp_rest(projL, t, p)
        if l >= 2:
            t = t.reshape(2 * t.shape[0], 4 * _F)
        xs.append(t)
        xs = _gin(xs, g, ginWx4, ginWg, ginb4, ginL, e)
    for _ in range(_POST):
        g = _dyn_hlvs(xs, pre, post, e)
        xs = _gin(xs, g, ginWx4, ginWg, ginb4, ginL, e)

    for k in range(_LEVELS):
        xk = xs[k]
        p = min(2 ** k, 4)
        if p > 1:
            parts = [xk[:, _F * i:_F * (i + 1)] for i in range(p)]
            xk = jnp.stack(parts, axis=1).reshape(p * xk.shape[0], _F)
        outs[k][...] = xk[:, :_N_FEAT_OUT]


def _bd4(W):
    a, b = W.shape
    z = jnp.zeros((a, b), W.dtype)
    rows = []
    for i in range(4):
        blocks = [z] * 4
        blocks[i] = W
        rows.append(jnp.concatenate(blocks, axis=1))
    return jnp.concatenate(rows, axis=0)


def _t4(b):
    return jnp.tile(b.reshape(1, -1), (1, 4))


def kernel(random_vector, params):
    x0 = random_vector.reshape(_N_EVENTS, _F)
    flat = []
    for W, b in params['pre_nn']:
        flat += [_bd4(W), _t4(b)]
    for W, b in params['post_nn']:
        flat += [W, b.reshape(1, -1)]
    for name in ('proj_nn', 'gin_nn'):
        (W0, b0), rest = params[name][0], params[name][1:]
        flat += [_bd4(W0[:_F]), W0[_F:], _t4(b0)]
        for W, b in rest:
            flat += [_bd4(W), _t4(b)]

    nblk = _N_EVENTS // _EB
    in_specs = [pl.BlockSpec((_EB, _F), lambda i: (i, 0))]
    in_specs += [pl.BlockSpec(w.shape, lambda i: (0, 0)) for w in flat]
    out_specs = [
        pl.BlockSpec((_EB * 2 ** l, _N_FEAT_OUT), lambda i, l=l: (i, 0))
        for l in range(_LEVELS)
    ]
    out_shape = [
        jax.ShapeDtypeStruct((_N_EVENTS * 2 ** l, _N_FEAT_OUT), jnp.float32)
        for l in range(_LEVELS)
    ]
    outs = pl.pallas_call(
        _tree_kernel,
        grid=(nblk,),
        in_specs=in_specs,
        out_specs=out_specs,
        out_shape=out_shape,
        compiler_params=pltpu.CompilerParams(
            dimension_semantics=("parallel",)),
    )(x0, *flat)
    return jnp.concatenate(outs, axis=0)


# speedup vs baseline: 9.9197x; 1.4060x over previous
from functools import partial

import jax
import jax.numpy as jnp
from jax.experimental import pallas as pl
from jax.experimental.pallas import tpu as pltpu

_N_EVENTS = 512
_N_FEAT_OUT = 3
_F = 32
_G = 8
_LEVELS = 8
_POST = 2
_EB = 128

_PRE_AB = [(_F, _F)] * 5
_PROJ_AB = [(_F + _G, _F + _G), (_F + _G, 2 * _F)] + [(2 * _F, 2 * _F)] * 3
_GIN_AB = [(_F + _G, _F + _G), (_F + _G, _F)] + [(_F, _F)] * 3


def _leaky(x):
    return jnp.maximum(x, 0.2 * x)


def _dot(x, w):
    return jnp.dot(x, w, preferred_element_type=jnp.float32)


def _pmlp_rest(layers, x, p):
    for W4, b4, a, b in layers:
        x = _leaky(_dot(x, W4[:p * a, :p * b]) + b4[:, :p * b])
    return x


def _pexpand(v, k, e):
    p = min(2 ** k, 4)
    if p > 1:
        v = jnp.concatenate([v] * p, axis=1)
    if k >= 3:
        r = 2 ** (k - 2)
        c = v.shape[1]
        v = jnp.broadcast_to(v[:, None, :], (e, r, c)).reshape(e * r, c)
    return v


def _fold(h, p):
    acc = h[:, :_F]
    for i in range(1, p):
        acc = acc + h[:, _F * i:_F * (i + 1)]
    return acc


def _dyn_hlvs(xs, pre, post, e):
    s = None
    for k, xk in enumerate(xs):
        p = min(2 ** k, 4)
        h = _pmlp_rest(pre, xk, p)
        if k >= 3:
            h = h.reshape(e, 2 ** (k - 2), 4 * _F).sum(axis=1)
        hk = _fold(h, p)
        s = hk if s is None else s + hk
    cnt = float(2 ** len(xs) - 1)
    means = s / cnt
    W0, b0 = post[0][0], post[0][1]
    t = _leaky(_dot(means, W0[:_F]) + _dot(s, W0[_F:]) + b0)
    for W, b, _, _ in post[1:]:
        t = _leaky(_dot(t, W) + b)
    return t


def _dup2(prev, p):
    if p == 1:
        return jnp.concatenate([prev, prev], axis=1)
    if p == 2:
        a, b = prev[:, :_F], prev[:, _F:]
        return jnp.concatenate([a, a, b, b], axis=1)
    gs = [prev[:, _F * i:_F * (i + 1)] for i in range(4)]
    d = jnp.concatenate(
        [gs[0], gs[0], gs[1], gs[1], gs[2], gs[2], gs[3], gs[3]], axis=1)
    return d.reshape(2 * prev.shape[0], 4 * _F)


def _gin(xs, g, ginWx4, ginWg, ginb4, ginL, e):
    gg = _dot(g, ginWg)
    new = []
    for k, xk in enumerate(xs):
        p = min(2 ** k, 4)
        if k == 0:
            u = xk
            gt = gg
        else:
            u = xk + _dup2(xs[k - 1], min(2 ** (k - 1), 4))
            gt = 2.0 * _pexpand(gg, k, e)
        w = _F + _G
        t = _leaky(_dot(u, ginWx4[:_F * p, :w * p]) + gt + ginb4[:, :w * p])
        new.append(_pmlp_rest(ginL, t, p))
    return new


def _tree_kernel(x0_ref, *refs):
    it = iter(refs[:42])

    def layers(n, ab):
        out = []
        for i in range(n):
            W = next(it)[...]
            b = next(it)[...]
            out.append((W, b, ab[i][0], ab[i][1]))
        return out

    pre = layers(5, _PRE_AB)
    post = layers(5, [(0, 0)] * 5)
    projWx4 = next(it)[...]
    projWg = next(it)[...]
    projb4 = next(it)[...]
    projL = layers(4, _PROJ_AB[1:])
    ginWx4 = next(it)[...]
    ginWg = next(it)[...]
    ginb4 = next(it)[...]
    ginL = layers(4, _GIN_AB[1:])
    outs = refs[42:]
    e = x0_ref.shape[0]

    xs = [x0_ref[...]]
    for l in range(_LEVELS - 1):
        g = _dyn_hlvs(xs, pre, post, e)
        p = min(2 ** l, 4)
        w = _F + _G
        gp = _pexpand(_dot(g, projWg), l, e)
        t = _leaky(_dot(xs[l], projWx4[:_F * p, :w * p]) + gp
                   + projb4[:, :w * p])
        t = _pmlp_rest(projL, t, p)
        if l >= 2:
            t = t.reshape(2 * t.shape[0], 4 * _F)
        xs.append(t)
        xs = _gin(xs, g, ginWx4, ginWg, ginb4, ginL, e)
    for _ in range(_POST):
        g = _dyn_hlvs(xs, pre, post, e)
        xs = _gin(xs, g, ginWx4, ginWg, ginb4, ginL, e)

    for k in range(_LEVELS):
        xk = xs[k]
        p = min(2 ** k, 4)
        if p > 1:
            parts = [xk[:, _F * i:_F * (i + 1)] for i in range(p)]
            xk = jnp.stack(parts, axis=1).reshape(p * xk.shape[0], _F)
        outs[k][...] = xk[:, :_N_FEAT_OUT]


def _bd4(W):
    a, b = W.shape
    z = jnp.zeros((a, b), W.dtype)
    rows = []
    for i in range(4):
        blocks = [z] * 4
        blocks[i] = W
        rows.append(jnp.concatenate(blocks, axis=1))
    return jnp.concatenate(rows, axis=0)


def _t4(b):
    return jnp.tile(b.reshape(1, -1), (1, 4))


def kernel(random_vector, params):
    x0 = random_vector.reshape(_N_EVENTS, _F)
    flat = []
    for W, b in params['pre_nn']:
        flat += [_bd4(W), _t4(b)]
    for W, b in params['post_nn']:
        flat += [W, b.reshape(1, -1)]
    for name in ('proj_nn', 'gin_nn'):
        (W0, b0), rest = params[name][0], params[name][1:]
        flat += [_bd4(W0[:_F]), W0[_F:], _t4(b0)]
        for W, b in rest:
            flat += [_bd4(W), _t4(b)]

    nblk = _N_EVENTS // _EB
    in_specs = [pl.BlockSpec((_EB, _F), lambda i: (i, 0))]
    in_specs += [pl.BlockSpec(w.shape, lambda i: (0, 0)) for w in flat]
    out_specs = [
        pl.BlockSpec((_EB * 2 ** l, _N_FEAT_OUT), lambda i, l=l: (i, 0))
        for l in range(_LEVELS)
    ]
    out_shape = [
        jax.ShapeDtypeStruct((_N_EVENTS * 2 ** l, _N_FEAT_OUT), jnp.float32)
        for l in range(_LEVELS)
    ]
    outs = pl.pallas_call(
        _tree_kernel,
        grid=(nblk,),
        in_specs=in_specs,
        out_specs=out_specs,
        out_shape=out_shape,
        compiler_params=pltpu.CompilerParams(
            dimension_semantics=("parallel",)),
    )(x0, *flat)
    return jnp.concatenate(outs, axis=0)


# speedup vs baseline: 13.6998x; 1.3811x over previous
from functools import partial

import jax
import jax.numpy as jnp
from jax.experimental import pallas as pl
from jax.experimental.pallas import tpu as pltpu

_N_EVENTS = 512
_N_FEAT_OUT = 3
_F = 32
_G = 8
_LEVELS = 8
_POST = 2
_EB = 256

_PRE_AB = [(_F, _F)] * 5
_PROJ_AB = [(_F + _G, _F + _G), (_F + _G, 2 * _F)] + [(2 * _F, 2 * _F)] * 3
_GIN_AB = [(_F + _G, _F + _G), (_F + _G, _F)] + [(_F, _F)] * 3


def _leaky(x):
    return jnp.maximum(x, 0.2 * x)


def _dot(x, w):
    return jnp.dot(x, w, preferred_element_type=jnp.float32)


def _pmlp_rest(layers, x, p):
    for W4, b4, a, b in layers:
        x = _leaky(_dot(x, W4[:p * a, :p * b]) + b4[:, :p * b])
    return x


def _pexpand(v, k, e):
    p = min(2 ** k, 4)
    if p > 1:
        v = jnp.concatenate([v] * p, axis=1)
    if k >= 3:
        r = 2 ** (k - 2)
        c = v.shape[1]
        v = jnp.broadcast_to(v[:, None, :], (e, r, c)).reshape(e * r, c)
    return v


def _fold(h, p):
    acc = h[:, :_F]
    for i in range(1, p):
        acc = acc + h[:, _F * i:_F * (i + 1)]
    return acc


def _dyn_hlvs(xs, pre, post, e):
    s = None
    for k, xk in enumerate(xs):
        p = min(2 ** k, 4)
        h = _pmlp_rest(pre, xk, p)
        if k >= 3:
            h = h.reshape(e, 2 ** (k - 2), 4 * _F).sum(axis=1)
        hk = _fold(h, p)
        s = hk if s is None else s + hk
    cnt = float(2 ** len(xs) - 1)
    means = s / cnt
    W0, b0 = post[0][0], post[0][1]
    t = _leaky(_dot(means, W0[:_F]) + _dot(s, W0[_F:]) + b0)
    for W, b, _, _ in post[1:]:
        t = _leaky(_dot(t, W) + b)
    return t


def _dup2(prev, p):
    if p == 1:
        return jnp.concatenate([prev, prev], axis=1)
    if p == 2:
        a, b = prev[:, :_F], prev[:, _F:]
        return jnp.concatenate([a, a, b, b], axis=1)
    gs = [prev[:, _F * i:_F * (i + 1)] for i in range(4)]
    d = jnp.concatenate(
        [gs[0], gs[0], gs[1], gs[1], gs[2], gs[2], gs[3], gs[3]], axis=1)
    return d.reshape(2 * prev.shape[0], 4 * _F)


def _gin(xs, g, ginWx4, ginWg, ginb4, ginL, e):
    gg = _dot(g, ginWg)
    new = []
    for k, xk in enumerate(xs):
        p = min(2 ** k, 4)
        if k == 0:
            u = xk
            gt = gg
        else:
            u = xk + _dup2(xs[k - 1], min(2 ** (k - 1), 4))
            gt = 2.0 * _pexpand(gg, k, e)
        w = _F + _G
        t = _leaky(_dot(u, ginWx4[:_F * p, :w * p]) + gt + ginb4[:, :w * p])
        new.append(_pmlp_rest(ginL, t, p))
    return new


def _tree_kernel(x0_ref, *refs):
    it = iter(refs[:42])

    def layers(n, ab):
        out = []
        for i in range(n):
            W = next(it)[...]
            b = next(it)[...]
            out.append((W, b, ab[i][0], ab[i][1]))
        return out

    pre = layers(5, _PRE_AB)
    post = layers(5, [(0, 0)] * 5)
    projWx4 = next(it)[...]
    projWg = next(it)[...]
    projb4 = next(it)[...]
    projL = layers(4, _PROJ_AB[1:])
    ginWx4 = next(it)[...]
    ginWg = next(it)[...]
    ginb4 = next(it)[...]
    ginL = layers(4, _GIN_AB[1:])
    outs = refs[42:]
    e = x0_ref.shape[0]

    xs = [x0_ref[...]]
    for l in range(_LEVELS - 1):
        g = _dyn_hlvs(xs, pre, post, e)
        p = min(2 ** l, 4)
        w = _F + _G
        gp = _pexpand(_dot(g, projWg), l, e)
        t = _leaky(_dot(xs[l], projWx4[:_F * p, :w * p]) + gp
                   + projb4[:, :w * p])
        t = _pmlp_rest(projL, t, p)
        if l >= 2:
            t = t.reshape(2 * t.shape[0], 4 * _F)
        xs.append(t)
        xs = _gin(xs, g, ginWx4, ginWg, ginb4, ginL, e)
    for _ in range(_POST):
        g = _dyn_hlvs(xs, pre, post, e)
        xs = _gin(xs, g, ginWx4, ginWg, ginb4, ginL, e)

    for k in range(_LEVELS):
        outs[k][...] = xs[k]


def _bd4(W):
    a, b = W.shape
    z = jnp.zeros((a, b), W.dtype)
    rows = []
    for i in range(4):
        blocks = [z] * 4
        blocks[i] = W
        rows.append(jnp.concatenate(blocks, axis=1))
    return jnp.concatenate(rows, axis=0)


def _t4(b):
    return jnp.tile(b.reshape(1, -1), (1, 4))


def kernel(random_vector, params):
    x0 = random_vector.reshape(_N_EVENTS, _F)
    flat = []
    for W, b in params['pre_nn']:
        flat += [_bd4(W), _t4(b)]
    for W, b in params['post_nn']:
        flat += [W, b.reshape(1, -1)]
    for name in ('proj_nn', 'gin_nn'):
        (W0, b0), rest = params[name][0], params[name][1:]
        flat += [_bd4(W0[:_F]), W0[_F:], _t4(b0)]
        for W, b in rest:
            flat += [_bd4(W), _t4(b)]

    nblk = _N_EVENTS // _EB
    in_specs = [pl.BlockSpec((_EB, _F), lambda i: (i, 0))]
    in_specs += [pl.BlockSpec(w.shape, lambda i: (0, 0)) for w in flat]
    packed_rows = [_N_EVENTS * 2 ** max(l - 2, 0) for l in range(_LEVELS)]
    packed_cols = [_F * min(2 ** l, 4) for l in range(_LEVELS)]
    out_specs = [
        pl.BlockSpec((packed_rows[l] // nblk, packed_cols[l]),
                     lambda i, l=l: (i, 0))
        for l in range(_LEVELS)
    ]
    out_shape = [
        jax.ShapeDtypeStruct((packed_rows[l], packed_cols[l]), jnp.float32)
        for l in range(_LEVELS)
    ]
    outs = pl.pallas_call(
        _tree_kernel,
        grid=(nblk,),
        in_specs=in_specs,
        out_specs=out_specs,
        out_shape=out_shape,
        compiler_params=pltpu.CompilerParams(
            dimension_semantics=("parallel",)),
    )(x0, *flat)
    outs = [o.reshape(_N_EVENTS * 2 ** l, _F)[:, :_N_FEAT_OUT]
            for l, o in enumerate(outs)]
    return jnp.concatenate(outs, axis=0)


# speedup vs baseline: 17.0001x; 1.2409x over previous
from functools import partial

import jax
import jax.numpy as jnp
from jax.experimental import pallas as pl
from jax.experimental.pallas import tpu as pltpu

_N_EVENTS = 512
_N_FEAT_OUT = 3
_F = 32
_G = 8
_LEVELS = 8
_POST = 2
_EB = 512

_PRE_AB = [(_F, _F)] * 5
_PROJ_AB = [(_F + _G, _F + _G), (_F + _G, 2 * _F)] + [(2 * _F, 2 * _F)] * 3
_GIN_AB = [(_F + _G, _F + _G), (_F + _G, _F)] + [(_F, _F)] * 3


def _leaky(x):
    return jnp.maximum(x, 0.2 * x)


def _dot(x, w):
    return jnp.dot(x, w, preferred_element_type=jnp.float32)


def _pmlp_rest(layers, x, p):
    for W4, b4, a, b in layers:
        x = _leaky(_dot(x, W4[:p * a, :p * b]) + b4[:, :p * b])
    return x


def _pexpand(v, k, e):
    p = min(2 ** k, 4)
    if p > 1:
        v = jnp.concatenate([v] * p, axis=1)
    if k >= 3:
        r = 2 ** (k - 2)
        c = v.shape[1]
        v = jnp.broadcast_to(v[:, None, :], (e, r, c)).reshape(e * r, c)
    return v


def _fold(h, p):
    acc = h[:, :_F]
    for i in range(1, p):
        acc = acc + h[:, _F * i:_F * (i + 1)]
    return acc


def _dyn_hlvs(xs, pre, post, e):
    s = None
    for k, xk in enumerate(xs):
        p = min(2 ** k, 4)
        h = _pmlp_rest(pre, xk, p)
        if k >= 3:
            h = h.reshape(e, 2 ** (k - 2), 4 * _F).sum(axis=1)
        hk = _fold(h, p)
        s = hk if s is None else s + hk
    cnt = float(2 ** len(xs) - 1)
    means = s / cnt
    W0, b0 = post[0][0], post[0][1]
    t = _leaky(_dot(means, W0[:_F]) + _dot(s, W0[_F:]) + b0)
    for W, b, _, _ in post[1:]:
        t = _leaky(_dot(t, W) + b)
    return t


def _dup2(prev, p):
    if p == 1:
        return jnp.concatenate([prev, prev], axis=1)
    if p == 2:
        a, b = prev[:, :_F], prev[:, _F:]
        return jnp.concatenate([a, a, b, b], axis=1)
    gs = [prev[:, _F * i:_F * (i + 1)] for i in range(4)]
    d = jnp.concatenate(
        [gs[0], gs[0], gs[1], gs[1], gs[2], gs[2], gs[3], gs[3]], axis=1)
    return d.reshape(2 * prev.shape[0], 4 * _F)


def _gin(xs, g, ginWx4, ginWg, ginb4, ginL, e):
    gg = _dot(g, ginWg)
    new = []
    for k, xk in enumerate(xs):
        p = min(2 ** k, 4)
        if k == 0:
            u = xk
            gt = gg
        else:
            u = xk + _dup2(xs[k - 1], min(2 ** (k - 1), 4))
            gt = 2.0 * _pexpand(gg, k, e)
        w = _F + _G
        t = _leaky(_dot(u, ginWx4[:_F * p, :w * p]) + gt + ginb4[:, :w * p])
        new.append(_pmlp_rest(ginL, t, p))
    return new


def _tree_kernel(x0_ref, *refs):
    it = iter(refs[:42])

    def layers(n, ab):
        out = []
        for i in range(n):
            W = next(it)[...]
            b = next(it)[...]
            out.append((W, b, ab[i][0], ab[i][1]))
        return out

    pre = layers(5, _PRE_AB)
    post = layers(5, [(0, 0)] * 5)
    projWx4 = next(it)[...]
    projWg = next(it)[...]
    projb4 = next(it)[...]
    projL = layers(4, _PROJ_AB[1:])
    ginWx4 = next(it)[...]
    ginWg = next(it)[...]
    ginb4 = next(it)[...]
    ginL = layers(4, _GIN_AB[1:])
    outs = refs[42:]
    e = x0_ref.shape[0]

    xs = [x0_ref[...]]
    for l in range(_LEVELS - 1):
        g = _dyn_hlvs(xs, pre, post, e)
        p = min(2 ** l, 4)
        w = _F + _G
        gp = _pexpand(_dot(g, projWg), l, e)
        t = _leaky(_dot(xs[l], projWx4[:_F * p, :w * p]) + gp
                   + projb4[:, :w * p])
        t = _pmlp_rest(projL, t, p)
        if l >= 2:
            t = t.reshape(2 * t.shape[0], 4 * _F)
        xs.append(t)
        xs = _gin(xs, g, ginWx4, ginWg, ginb4, ginL, e)
    for _ in range(_POST):
        g = _dyn_hlvs(xs, pre, post, e)
        xs = _gin(xs, g, ginWx4, ginWg, ginb4, ginL, e)

    for k in range(_LEVELS):
        outs[k][...] = xs[k]


def _bd4(W):
    a, b = W.shape
    z = jnp.zeros((a, b), W.dtype)
    rows = []
    for i in range(4):
        blocks = [z] * 4
        blocks[i] = W
        rows.append(jnp.concatenate(blocks, axis=1))
    return jnp.concatenate(rows, axis=0)


def _t4(b):
    return jnp.tile(b.reshape(1, -1), (1, 4))


def kernel(random_vector, params):
    x0 = random_vector.reshape(_N_EVENTS, _F)
    flat = []
    for W, b in params['pre_nn']:
        flat += [_bd4(W), _t4(b)]
    for W, b in params['post_nn']:
        flat += [W, b.reshape(1, -1)]
    for name in ('proj_nn', 'gin_nn'):
        (W0, b0), rest = params[name][0], params[name][1:]
        flat += [_bd4(W0[:_F]), W0[_F:], _t4(b0)]
        for W, b in rest:
            flat += [_bd4(W), _t4(b)]

    nblk = _N_EVENTS // _EB
    in_specs = [pl.BlockSpec((_EB, _F), lambda i: (i, 0))]
    in_specs += [pl.BlockSpec(w.shape, lambda i: (0, 0)) for w in flat]
    packed_rows = [_N_EVENTS * 2 ** max(l - 2, 0) for l in range(_LEVELS)]
    packed_cols = [_F * min(2 ** l, 4) for l in range(_LEVELS)]
    out_specs = [
        pl.BlockSpec((packed_rows[l] // nblk, packed_cols[l]),
                     lambda i, l=l: (i, 0))
        for l in range(_LEVELS)
    ]
    out_shape = [
        jax.ShapeDtypeStruct((packed_rows[l], packed_cols[l]), jnp.float32)
        for l in range(_LEVELS)
    ]
    outs = pl.pallas_call(
        _tree_kernel,
        grid=(nblk,),
        in_specs=in_specs,
        out_specs=out_specs,
        out_shape=out_shape,
        compiler_params=pltpu.CompilerParams(
            dimension_semantics=("parallel",)),
    )(x0, *flat)
    outs = [o.reshape(_N_EVENTS * 2 ** l, _F)[:, :_N_FEAT_OUT]
            for l, o in enumerate(outs)]
    return jnp.concatenate(outs, axis=0)


# speedup vs baseline: 17.2100x; 1.0123x over previous
from functools import partial

import jax
import jax.numpy as jnp
from jax.experimental import pallas as pl
from jax.experimental.pallas import tpu as pltpu

_N_EVENTS = 512
_N_FEAT_OUT = 3
_F = 32
_G = 8
_LEVELS = 8
_POST = 2
_EB = 512

_PRE_AB = [(_F, _F)] * 5
_PROJ_AB = [(_F + _G, _F + _G), (_F + _G, 2 * _F)] + [(2 * _F, 2 * _F)] * 3
_GIN_AB = [(_F + _G, _F + _G), (_F + _G, _F)] + [(_F, _F)] * 3


def _leaky(x):
    return jnp.maximum(x, 0.2 * x)


def _dot(x, w):
    return jnp.dot(x, w, preferred_element_type=jnp.float32)


def _pmlp_rest(layers, x, p):
    for W4, b4, a, b in layers:
        x = _leaky(_dot(x, W4[:p * a, :p * b]) + b4[:, :p * b])
    return x


def _pexpand(v, k, e):
    p = min(2 ** k, 4)
    if p > 1:
        v = jnp.concatenate([v] * p, axis=1)
    if k >= 3:
        r = 2 ** (k - 2)
        c = v.shape[1]
        v = jnp.broadcast_to(v[:, None, :], (e, r, c)).reshape(e * r, c)
    return v


def _fold(h, p):
    acc = h[:, :_F]
    for i in range(1, p):
        acc = acc + h[:, _F * i:_F * (i + 1)]
    return acc


def _dyn_hlvs(xs, pre, post, e):
    s = None
    for k, xk in enumerate(xs):
        p = min(2 ** k, 4)
        h = _pmlp_rest(pre, xk, p)
        if k >= 3:
            h = h.reshape(e, 2 ** (k - 2), 4 * _F).sum(axis=1)
        hk = _fold(h, p)
        s = hk if s is None else s + hk
    cnt = float(2 ** len(xs) - 1)
    W0, b0 = post[0][0], post[0][1]
    t = _leaky(_dot(s, W0[:_F] * (1.0 / cnt) + W0[_F:]) + b0)
    for W, b, _, _ in post[1:]:
        t = _leaky(_dot(t, W) + b)
    return t


def _dup2(prev, p):
    if p == 1:
        return jnp.concatenate([prev, prev], axis=1)
    if p == 2:
        a, b = prev[:, :_F], prev[:, _F:]
        return jnp.concatenate([a, a, b, b], axis=1)
    gs = [prev[:, _F * i:_F * (i + 1)] for i in range(4)]
    d = jnp.concatenate(
        [gs[0], gs[0], gs[1], gs[1], gs[2], gs[2], gs[3], gs[3]], axis=1)
    return d.reshape(2 * prev.shape[0], 4 * _F)


def _gin(xs, g, ginWx4, ginWg, ginb4, ginL, e):
    gg = _dot(g, ginWg)
    w = _F + _G
    gg1 = gg + ginb4[:, :w]
    gg2 = 2.0 * gg + ginb4[:, :w]
    new = []
    for k, xk in enumerate(xs):
        p = min(2 ** k, 4)
        if k == 0:
            u = xk
            gt = gg1
        else:
            u = xk + _dup2(xs[k - 1], min(2 ** (k - 1), 4))
            gt = _pexpand(gg2, k, e)
        t = _leaky(_dot(u, ginWx4[:_F * p, :w * p]) + gt)
        new.append(_pmlp_rest(ginL, t, p))
    return new


def _tree_kernel(x0_ref, *refs):
    it = iter(refs[:42])

    def layers(n, ab):
        out = []
        for i in range(n):
            W = next(it)[...]
            b = next(it)[...]
            out.append((W, b, ab[i][0], ab[i][1]))
        return out

    pre = layers(5, _PRE_AB)
    post = layers(5, [(0, 0)] * 5)
    projWx4 = next(it)[...]
    projWg = next(it)[...]
    projb4 = next(it)[...]
    projL = layers(4, _PROJ_AB[1:])
    ginWx4 = next(it)[...]
    ginWg = next(it)[...]
    ginb4 = next(it)[...]
    ginL = layers(4, _GIN_AB[1:])
    outs = refs[42:]
    e = x0_ref.shape[0]

    xs = [x0_ref[...]]
    for l in range(_LEVELS - 1):
        g = _dyn_hlvs(xs, pre, post, e)
        p = min(2 ** l, 4)
        w = _F + _G
        gp = _pexpand(_dot(g, projWg) + projb4[:, :w], l, e)
        t = _leaky(_dot(xs[l], projWx4[:_F * p, :w * p]) + gp)
        t = _pmlp_rest(projL, t, p)
        if l >= 2:
            t = t.reshape(2 * t.shape[0], 4 * _F)
        xs.append(t)
        xs = _gin(xs, g, ginWx4, ginWg, ginb4, ginL, e)
    for _ in range(_POST):
        g = _dyn_hlvs(xs, pre, post, e)
        xs = _gin(xs, g, ginWx4, ginWg, ginb4, ginL, e)

    for k in range(_LEVELS):
        outs[k][...] = xs[k]


def _bd4(W):
    a, b = W.shape
    z = jnp.zeros((a, b), W.dtype)
    rows = []
    for i in range(4):
        blocks = [z] * 4
        blocks[i] = W
        rows.append(jnp.concatenate(blocks, axis=1))
    return jnp.concatenate(rows, axis=0)


def _t4(b):
    return jnp.tile(b.reshape(1, -1), (1, 4))


def kernel(random_vector, params):
    x0 = random_vector.reshape(_N_EVENTS, _F)
    flat = []
    for W, b in params['pre_nn']:
        flat += [_bd4(W), _t4(b)]
    for W, b in params['post_nn']:
        flat += [W, b.reshape(1, -1)]
    for name in ('proj_nn', 'gin_nn'):
        (W0, b0), rest = params[name][0], params[name][1:]
        flat += [_bd4(W0[:_F]), W0[_F:], _t4(b0)]
        for W, b in rest:
            flat += [_bd4(W), _t4(b)]

    nblk = _N_EVENTS // _EB
    in_specs = [pl.BlockSpec((_EB, _F), lambda i: (i, 0))]
    in_specs += [pl.BlockSpec(w.shape, lambda i: (0, 0)) for w in flat]
    packed_rows = [_N_EVENTS * 2 ** max(l - 2, 0) for l in range(_LEVELS)]
    packed_cols = [_F * min(2 ** l, 4) for l in range(_LEVELS)]
    out_specs = [
        pl.BlockSpec((packed_rows[l] // nblk, packed_cols[l]),
                     lambda i, l=l: (i, 0))
        for l in range(_LEVELS)
    ]
    out_shape = [
        jax.ShapeDtypeStruct((packed_rows[l], packed_cols[l]), jnp.float32)
        for l in range(_LEVELS)
    ]
    outs = pl.pallas_call(
        _tree_kernel,
        grid=(nblk,),
        in_specs=in_specs,
        out_specs=out_specs,
        out_shape=out_shape,
        compiler_params=pltpu.CompilerParams(
            dimension_semantics=("parallel",)),
    )(x0, *flat)
    outs = [o.reshape(_N_EVENTS * 2 ** l, _F)[:, :_N_FEAT_OUT]
            for l, o in enumerate(outs)]
    return jnp.concatenate(outs, axis=0)
